# Initial kernel scaffold; baseline (speedup 1.0000x reference)
import jax
import jax.numpy as jnp
from jax.experimental import pallas as pl


def kernel(x, edge_index, W1, b1, W2, b2, head_w, head_b):
    raise NotImplementedError("write your pallas kernel here")


# speedup vs baseline: 5.7479x; 5.7479x over previous
import functools

import jax
import jax.numpy as jnp
from jax import lax
from jax.experimental import pallas as pl
from jax.experimental.pallas import tpu as pltpu
from jax.experimental.pallas import tpu_sc as plsc

N_NODES = 10000
N_PAD = 10240
D = 128
E = 320000
NC, NS = 2, 16
NW = NC * NS
CHUNK = 128
CH_PER_W = (E // NW + CHUNK - 1) // CHUNK
EDGES_PER_W = CH_PER_W * CHUNK
E_PAD = EDGES_PER_W * NW
ROWS_PER_TILE = N_PAD // NS


def _seg_sum_body(x_hbm, src_hbm, dst_hbm, out_hbm,
                  idx_src, idx_dst, rows, acc, sem):
    c = lax.axis_index("c")
    s = lax.axis_index("s")
    w = c * NS + s

    zeros16 = jnp.zeros((16,), jnp.float32)

    def zero_body(i, _):
        r = i // (D // 16)
        col = (i % (D // 16)) * 16
        rows[r, pl.ds(col, 16)] = zeros16
        return 0

    lax.fori_loop(0, CHUNK * (D // 16), zero_body, 0)
    for t in range(ROWS_PER_TILE // CHUNK):
        pltpu.sync_copy(rows, acc.at[pl.ds(s * ROWS_PER_TILE + t * CHUNK,
                                           CHUNK)])
    plsc.subcore_barrier()

    def body(j, _):
        base = pl.multiple_of(w * EDGES_PER_W + j * CHUNK, 8)
        pltpu.sync_copy(src_hbm.at[pl.ds(base, CHUNK)], idx_src)
        pltpu.sync_copy(dst_hbm.at[pl.ds(base, CHUNK)], idx_dst)
        pltpu.async_copy(x_hbm.at[idx_src], rows, sem).wait()
        pltpu.sync_copy(rows, acc.at[idx_dst], add=True)
        return 0

    lax.fori_loop(0, CH_PER_W, body, 0)
    plsc.subcore_barrier()

    for t in range(ROWS_PER_TILE // CHUNK):
        r0 = s * ROWS_PER_TILE + t * CHUNK
        pltpu.sync_copy(acc.at[pl.ds(r0, CHUNK)], rows)
        pltpu.sync_copy(rows, out_hbm.at[c, pl.ds(r0, CHUNK)])


_seg_sum = pl.kernel(
    _seg_sum_body,
    out_type=jax.ShapeDtypeStruct((NC, N_PAD, D), jnp.float32),
    mesh=plsc.VectorSubcoreMesh(core_axis_name="c", subcore_axis_name="s"),
    scratch_types=[
        pltpu.VMEM((CHUNK,), jnp.int32),
        pltpu.VMEM((CHUNK,), jnp.int32),
        pltpu.VMEM((CHUNK, D), jnp.float32),
        pltpu.VMEM_SHARED((N_PAD, D), jnp.float32),
        pltpu.SemaphoreType.DMA,
    ],
)


def _layer1_body(p_ref, W_ref, b_ref, o_ref):
    a = p_ref[0] + p_ref[1]
    o_ref[...] = jnp.maximum(
        jnp.dot(a, W_ref[...], preferred_element_type=jnp.float32)
        + b_ref[...], 0.0)


def _layer2_body(p_ref, W_ref, b_ref, hw_ref, hb_ref, o_ref):
    a = p_ref[0] + p_ref[1]
    h = jnp.maximum(
        jnp.dot(a, W_ref[...], preferred_element_type=jnp.float32)
        + b_ref[...], 0.0)
    o_ref[...] = jnp.dot(h, hw_ref[...],
                         preferred_element_type=jnp.float32) + hb_ref[...]


_BN = 1024


def _tc_layer1(p, W, b):
    return pl.pallas_call(
        _layer1_body,
        grid=(N_PAD // _BN,),
        in_specs=[
            pl.BlockSpec((NC, _BN, D), lambda i: (0, i, 0)),
            pl.BlockSpec((D, D), lambda i: (0, 0)),
            pl.BlockSpec((1, D), lambda i: (0, 0)),
        ],
        out_specs=pl.BlockSpec((_BN, D), lambda i: (i, 0)),
        out_shape=jax.ShapeDtypeStruct((N_PAD, D), jnp.float32),
    )(p, W, b)


def _tc_layer2(p, W, b, head_w, head_b):
    return pl.pallas_call(
        _layer2_body,
        grid=(N_PAD // _BN,),
        in_specs=[
            pl.BlockSpec((NC, _BN, D), lambda i: (0, i, 0)),
            pl.BlockSpec((D, D), lambda i: (0, 0)),
            pl.BlockSpec((1, D), lambda i: (0, 0)),
            pl.BlockSpec((D, 1), lambda i: (0, 0)),
            pl.BlockSpec((1, 1), lambda i: (0, 0)),
        ],
        out_specs=pl.BlockSpec((_BN, 1), lambda i: (i, 0)),
        out_shape=jax.ShapeDtypeStruct((N_PAD, 1), jnp.float32),
    )(p, W, b, head_w, head_b)


def kernel(x, edge_index, W1, b1, W2, b2, head_w, head_b):
    src = edge_index[0]
    dst = edge_index[1]
    pad = E_PAD - E
    it = lax.iota(jnp.int32, pad)
    src_p = jnp.concatenate([src, it % N_NODES])
    dst_p = jnp.concatenate([dst, N_NODES + it % (N_PAD - N_NODES)])

    agg1 = _seg_sum(x, src_p, dst_p)
    h1 = _tc_layer1(agg1, W1, b1.reshape(1, D))
    agg2 = _seg_sum(h1, src_p, dst_p)
    logits = _tc_layer2(agg2, W2, b2.reshape(1, D),
                        head_w, head_b.reshape(1, 1))
    return logits[:N_NODES, 0]


# speedup vs baseline: 8.0823x; 1.4061x over previous
import jax
import jax.numpy as jnp
from jax import lax
from jax.experimental import pallas as pl
from jax.experimental.pallas import tpu as pltpu
from jax.experimental.pallas import tpu_sc as plsc

N_NODES = 10000
N_PAD = 10240
D = 128
E = 320000
NC, NS = 2, 16
NW = NC * NS
CHUNK = 128
CH_PER_W = 80
EDGES_PER_W = CH_PER_W * CHUNK
E_PAD = EDGES_PER_W * NW
ROWS_PER_TILE = N_PAD // NS


def _seg_sum_body(x_hbm, src_hbm, dst_hbm, out_hbm,
                  srci0, srci1, dsti0, dsti1, rows0, rows1, acc,
                  sis0, sis1, dis0, dis1, gsem0, gsem1, ssem0, ssem1):
    c = lax.axis_index("c")
    s = lax.axis_index("s")
    w = c * NS + s

    zeros16 = jnp.zeros((16,), jnp.float32)

    def zero_body(i, _):
        r = i // (D // 16)
        col = (i % (D // 16)) * 16
        rows0[r, pl.ds(col, 16)] = zeros16
        return 0

    lax.fori_loop(0, CHUNK * (D // 16), zero_body, 0)
    for t in range(ROWS_PER_TILE // CHUNK):
        pltpu.sync_copy(rows0, acc.at[pl.ds(s * ROWS_PER_TILE + t * CHUNK,
                                            CHUNK)])
    plsc.subcore_barrier()

    def body(i, _):
        j = 2 * i
        b0 = pl.multiple_of(w * EDGES_PER_W + j * CHUNK, 8)
        b1 = pl.multiple_of(w * EDGES_PER_W + (j + 1) * CHUNK, 8)
        a0 = pltpu.async_copy(src_hbm.at[pl.ds(b0, CHUNK)], srci0, sis0)
        d0 = pltpu.async_copy(dst_hbm.at[pl.ds(b0, CHUNK)], dsti0, dis0)
        a1 = pltpu.async_copy(src_hbm.at[pl.ds(b1, CHUNK)], srci1, sis1)
        d1 = pltpu.async_copy(dst_hbm.at[pl.ds(b1, CHUNK)], dsti1, dis1)
        a0.wait()
        g0 = pltpu.async_copy(x_hbm.at[srci0], rows0, gsem0)
        a1.wait()
        g1 = pltpu.async_copy(x_hbm.at[srci1], rows1, gsem1)
        g0.wait()
        d0.wait()
        s0 = pltpu.async_copy(rows0, acc.at[dsti0], ssem0, add=True)
        g1.wait()
        d1.wait()
        s1 = pltpu.async_copy(rows1, acc.at[dsti1], ssem1, add=True)
        s0.wait()
        s1.wait()
        return 0

    lax.fori_loop(0, CH_PER_W // 2, body, 0)
    plsc.subcore_barrier()

    for t in range(ROWS_PER_TILE // CHUNK):
        r0 = s * ROWS_PER_TILE + t * CHUNK
        pltpu.sync_copy(acc.at[pl.ds(r0, CHUNK)], rows0)
        pltpu.sync_copy(rows0, out_hbm.at[c, pl.ds(r0, CHUNK)])


_seg_sum = pl.kernel(
    _seg_sum_body,
    out_type=jax.ShapeDtypeStruct((NC, N_PAD, D), jnp.float32),
    mesh=plsc.VectorSubcoreMesh(core_axis_name="c", subcore_axis_name="s"),
    scratch_types=[
        pltpu.VMEM((CHUNK,), jnp.int32),
        pltpu.VMEM((CHUNK,), jnp.int32),
        pltpu.VMEM((CHUNK,), jnp.int32),
        pltpu.VMEM((CHUNK,), jnp.int32),
        pltpu.VMEM((CHUNK, D), jnp.float32),
        pltpu.VMEM((CHUNK, D), jnp.float32),
        pltpu.VMEM_SHARED((N_PAD, D), jnp.float32),
    ] + [pltpu.SemaphoreType.DMA] * 8,
)


def _layer1_body(p_ref, W_ref, b_ref, o_ref):
    a = p_ref[0] + p_ref[1]
    o_ref[...] = jnp.maximum(
        jnp.dot(a, W_ref[...], preferred_element_type=jnp.float32)
        + b_ref[...], 0.0)


def _layer2_body(p_ref, W_ref, b_ref, hw_ref, hb_ref, o_ref):
    a = p_ref[0] + p_ref[1]
    h = jnp.maximum(
        jnp.dot(a, W_ref[...], preferred_element_type=jnp.float32)
        + b_ref[...], 0.0)
    o_ref[...] = jnp.dot(h, hw_ref[...],
                         preferred_element_type=jnp.float32) + hb_ref[...]


_BN = 1024


def _tc_layer1(p, W, b):
    return pl.pallas_call(
        _layer1_body,
        grid=(N_PAD // _BN,),
        in_specs=[
            pl.BlockSpec((NC, _BN, D), lambda i: (0, i, 0)),
            pl.BlockSpec((D, D), lambda i: (0, 0)),
            pl.BlockSpec((1, D), lambda i: (0, 0)),
        ],
        out_specs=pl.BlockSpec((_BN, D), lambda i: (i, 0)),
        out_shape=jax.ShapeDtypeStruct((N_PAD, D), jnp.float32),
    )(p, W, b)


def _tc_layer2(p, W, b, head_w, head_b):
    return pl.pallas_call(
        _layer2_body,
        grid=(N_PAD // _BN,),
        in_specs=[
            pl.BlockSpec((NC, _BN, D), lambda i: (0, i, 0)),
            pl.BlockSpec((D, D), lambda i: (0, 0)),
            pl.BlockSpec((1, D), lambda i: (0, 0)),
            pl.BlockSpec((D, 1), lambda i: (0, 0)),
            pl.BlockSpec((1, 1), lambda i: (0, 0)),
        ],
        out_specs=pl.BlockSpec((_BN, 1), lambda i: (i, 0)),
        out_shape=jax.ShapeDtypeStruct((N_PAD, 1), jnp.float32),
    )(p, W, b, head_w, head_b)


def kernel(x, edge_index, W1, b1, W2, b2, head_w, head_b):
    src = edge_index[0]
    dst = edge_index[1]
    pad = E_PAD - E
    it = lax.iota(jnp.int32, pad)
    src_p = jnp.concatenate([src, it % N_NODES])
    dst_p = jnp.concatenate([dst, N_NODES + it % (N_PAD - N_NODES)])

    agg1 = _seg_sum(x, src_p, dst_p)
    h1 = _tc_layer1(agg1, W1, b1.reshape(1, D))
    agg2 = _seg_sum(h1, src_p, dst_p)
    logits = _tc_layer2(agg2, W2, b2.reshape(1, D),
                        head_w, head_b.reshape(1, 1))
    return logits[:N_NODES, 0]


# speedup vs baseline: 8.9693x; 1.1097x over previous
import jax
import jax.numpy as jnp
from jax import lax
from jax.experimental import pallas as pl
from jax.experimental.pallas import tpu as pltpu
from jax.experimental.pallas import tpu_sc as plsc

N_NODES = 10000
N_PAD = 10240
D = 128
E = 320000
NC, NS = 2, 16
NW = NC * NS
CHUNK = 128
NSETS = 2
CH_PER_W = 80
NIT = CH_PER_W // NSETS
EDGES_PER_W = CH_PER_W * CHUNK
E_PAD = EDGES_PER_W * NW
ROWS_PER_TILE = N_PAD // NS


def _seg_sum_body(x_hbm, src_hbm, dst_hbm, out_hbm,
                  srci, dsti, rows0, rows1, acc,
                  sis, dis, gsem, ssem, zsem):
    c = lax.axis_index("c")
    s = lax.axis_index("s")
    w = c * NS + s
    rows = [rows0, rows1]

    zeros16 = jnp.zeros((16,), jnp.float32)

    def zero_body(i, _):
        r = i // (D // 16)
        col = (i % (D // 16)) * 16
        rows0[r, pl.ds(col, 16)] = zeros16
        return 0

    lax.fori_loop(0, CHUNK * (D // 16), zero_body, 0)
    zd = [pltpu.async_copy(
        rows0, acc.at[pl.ds(s * ROWS_PER_TILE + t * CHUNK, CHUNK)], zsem)
        for t in range(ROWS_PER_TILE // CHUNK)]
    for d in zd:
        d.wait()
    plsc.subcore_barrier()

    def idx_issue(bank, i):
        for k in range(NSETS):
            b = pl.multiple_of(
                w * EDGES_PER_W + (i * NSETS + k) * CHUNK, 8)
            kb = bank * NSETS + k
            pltpu.async_copy(src_hbm.at[pl.ds(b, CHUNK)], srci.at[kb],
                             sis.at[kb])
            pltpu.async_copy(dst_hbm.at[pl.ds(b, CHUNK)], dsti.at[kb],
                             dis.at[kb])

    def idx_wait(bank, i):
        for k in range(NSETS):
            b = pl.multiple_of(
                w * EDGES_PER_W + (i * NSETS + k) * CHUNK, 8)
            kb = bank * NSETS + k
            pltpu.make_async_copy(src_hbm.at[pl.ds(b, CHUNK)], srci.at[kb],
                                  sis.at[kb]).wait()

    idx_issue(0, 0)

    def body(i2, _):
        for bank in range(2):
            i = 2 * i2 + bank
            base = bank * NSETS
            g = []
            for k in range(NSETS):
                kb = base + k
                b = pl.multiple_of(
                    w * EDGES_PER_W + (i * NSETS + k) * CHUNK, 8)
                pltpu.make_async_copy(src_hbm.at[pl.ds(b, CHUNK)],
                                      srci.at[kb], sis.at[kb]).wait()
                g.append(pltpu.async_copy(x_hbm.at[srci.at[kb]], rows[k],
                                          gsem.at[k]))
            if bank == 0:
                idx_issue(1, i + 1)
            else:
                @pl.when(i2 < NIT // 2 - 1)
                def _():
                    idx_issue(0, i + 1)
            ss = []
            for k in range(NSETS):
                kb = base + k
                b = pl.multiple_of(
                    w * EDGES_PER_W + (i * NSETS + k) * CHUNK, 8)
                g[k].wait()
                pltpu.make_async_copy(dst_hbm.at[pl.ds(b, CHUNK)],
                                      dsti.at[kb], dis.at[kb]).wait()
                ss.append(pltpu.async_copy(rows[k], acc.at[dsti.at[kb]],
                                           ssem.at[k], add=True))
            for d in ss:
                d.wait()
        return 0

    lax.fori_loop(0, NIT // 2, body, 0)
    plsc.subcore_barrier()

    wd = []
    for t in range(ROWS_PER_TILE // CHUNK):
        k = t % NSETS
        r0 = s * ROWS_PER_TILE + t * CHUNK
        if t >= NSETS:
            wd[t - NSETS].wait()
        pltpu.sync_copy(acc.at[pl.ds(r0, CHUNK)], rows[k])
        wd.append(pltpu.async_copy(rows[k], out_hbm.at[c, pl.ds(r0, CHUNK)],
                                   ssem.at[k]))
    for t in range(max(0, ROWS_PER_TILE // CHUNK - NSETS),
                   ROWS_PER_TILE // CHUNK):
        wd[t].wait()


_seg_sum = pl.kernel(
    _seg_sum_body,
    out_type=jax.ShapeDtypeStruct((NC, N_PAD, D), jnp.float32),
    mesh=plsc.VectorSubcoreMesh(core_axis_name="c", subcore_axis_name="s"),
    scratch_types=[
        pltpu.VMEM((2 * NSETS, CHUNK), jnp.int32),
        pltpu.VMEM((2 * NSETS, CHUNK), jnp.int32),
        pltpu.VMEM((CHUNK, D), jnp.float32),
        pltpu.VMEM((CHUNK, D), jnp.float32),
        pltpu.VMEM_SHARED((N_PAD, D), jnp.float32),
        pltpu.SemaphoreType.DMA((2 * NSETS,)),
        pltpu.SemaphoreType.DMA((2 * NSETS,)),
        pltpu.SemaphoreType.DMA((NSETS,)),
        pltpu.SemaphoreType.DMA((NSETS,)),
        pltpu.SemaphoreType.DMA,
    ],
)


def _layer1_body(p_ref, W_ref, b_ref, o_ref):
    a = p_ref[0] + p_ref[1]
    o_ref[...] = jnp.maximum(
        jnp.dot(a, W_ref[...], preferred_element_type=jnp.float32)
        + b_ref[...], 0.0)


def _layer2_body(p_ref, W_ref, b_ref, hw_ref, hb_ref, o_ref):
    a = p_ref[0] + p_ref[1]
    h = jnp.maximum(
        jnp.dot(a, W_ref[...], preferred_element_type=jnp.float32)
        + b_ref[...], 0.0)
    o_ref[...] = jnp.dot(h, hw_ref[...],
                         preferred_element_type=jnp.float32) + hb_ref[...]


_BN = 1024


def _tc_layer1(p, W, b):
    return pl.pallas_call(
        _layer1_body,
        grid=(N_PAD // _BN,),
        in_specs=[
            pl.BlockSpec((NC, _BN, D), lambda i: (0, i, 0)),
            pl.BlockSpec((D, D), lambda i: (0, 0)),
            pl.BlockSpec((1, D), lambda i: (0, 0)),
        ],
        out_specs=pl.BlockSpec((_BN, D), lambda i: (i, 0)),
        out_shape=jax.ShapeDtypeStruct((N_PAD, D), jnp.float32),
    )(p, W, b)


def _tc_layer2(p, W, b, head_w, head_b):
    return pl.pallas_call(
        _layer2_body,
        grid=(N_PAD // _BN,),
        in_specs=[
            pl.BlockSpec((NC, _BN, D), lambda i: (0, i, 0)),
            pl.BlockSpec((D, D), lambda i: (0, 0)),
            pl.BlockSpec((1, D), lambda i: (0, 0)),
            pl.BlockSpec((D, 1), lambda i: (0, 0)),
            pl.BlockSpec((1, 1), lambda i: (0, 0)),
        ],
        out_specs=pl.BlockSpec((_BN, 1), lambda i: (i, 0)),
        out_shape=jax.ShapeDtypeStruct((N_PAD, 1), jnp.float32),
    )(p, W, b, head_w, head_b)


def kernel(x, edge_index, W1, b1, W2, b2, head_w, head_b):
    src = edge_index[0]
    dst = edge_index[1]
    pad = E_PAD - E
    it = lax.iota(jnp.int32, pad)
    src_p = jnp.concatenate([src, it % N_NODES])
    dst_p = jnp.concatenate([dst, N_NODES + it % (N_PAD - N_NODES)])

    agg1 = _seg_sum(x, src_p, dst_p)
    h1 = _tc_layer1(agg1, W1, b1.reshape(1, D))
    agg2 = _seg_sum(h1, src_p, dst_p)
    logits = _tc_layer2(agg2, W2, b2.reshape(1, D),
                        head_w, head_b.reshape(1, 1))
    return logits[:N_NODES, 0]


# speedup vs baseline: 11.7123x; 1.3058x over previous
import jax
import jax.numpy as jnp
from jax import lax
from jax.experimental import pallas as pl
from jax.experimental.pallas import tpu as pltpu
from jax.experimental.pallas import tpu_sc as plsc

N_NODES = 10000
N_PAD = 10240
D = 128
E = 320000
NC, NS = 2, 16
NW = NC * NS
CHUNK = 128
NSETS = 2
CH_PER_W = 80
NIT = CH_PER_W // NSETS
EDGES_PER_W = CH_PER_W * CHUNK
E_PAD = EDGES_PER_W * NW
ROWS_PER_TILE = N_PAD // NS


def _seg_sum_body(x_hbm, src_hbm, dst_hbm, out_hbm,
                  srci, dsti, rows0, rows1, acc,
                  sis, dis, gsem, ssem, zsem):
    c = lax.axis_index("c")
    s = lax.axis_index("s")
    w = c * NS + s
    rows = [rows0, rows1]

    zeros16 = jnp.zeros((16,), jnp.float32)

    def zero_body(i, _):
        r = i // (D // 16)
        col = (i % (D // 16)) * 16
        rows0[r, pl.ds(col, 16)] = zeros16
        return 0

    lax.fori_loop(0, CHUNK * (D // 16), zero_body, 0)
    zd = [pltpu.async_copy(
        rows0, acc.at[pl.ds(s * ROWS_PER_TILE + t * CHUNK, CHUNK)], zsem)
        for t in range(ROWS_PER_TILE // CHUNK)]
    for d in zd:
        d.wait()
    plsc.subcore_barrier()

    def idx_issue(slot, j):
        b = pl.multiple_of(w * EDGES_PER_W + j * CHUNK, 8)
        pltpu.async_copy(src_hbm.at[pl.ds(b, CHUNK)], srci.at[slot],
                         sis.at[slot])
        pltpu.async_copy(dst_hbm.at[pl.ds(b, CHUNK)], dsti.at[slot],
                         dis.at[slot])

    idx_issue(0, 0)
    idx_issue(1, 1)
    pltpu.make_async_copy(src_hbm.at[pl.ds(
        pl.multiple_of(w * EDGES_PER_W, 8), CHUNK)],
        srci.at[0], sis.at[0]).wait()
    pltpu.async_copy(x_hbm.at[srci.at[0]], rows[0], gsem.at[0])

    def body(i4, _):
        j0 = 4 * i4
        for t in range(4):
            j = j0 + t
            bank = t % 2
            slot = t
            b = pl.multiple_of(w * EDGES_PER_W + j * CHUNK, 8)

            @pl.when(j >= 1)
            def _():
                pltpu.make_async_copy(rows[1 - bank],
                                      acc.at[dsti.at[(t + 3) % 4]],
                                      ssem.at[1 - bank]).wait()

            @pl.when(j + 2 < CH_PER_W)
            def _():
                idx_issue((t + 2) % 4, j + 2)

            @pl.when(j + 1 < CH_PER_W)
            def _():
                b1 = pl.multiple_of(w * EDGES_PER_W + (j + 1) * CHUNK, 8)
                pltpu.make_async_copy(src_hbm.at[pl.ds(b1, CHUNK)],
                                      srci.at[(t + 1) % 4],
                                      sis.at[(t + 1) % 4]).wait()
                pltpu.async_copy(x_hbm.at[srci.at[(t + 1) % 4]],
                                 rows[1 - bank], gsem.at[1 - bank])

            pltpu.make_async_copy(x_hbm.at[srci.at[slot]], rows[bank],
                                  gsem.at[bank]).wait()
            pltpu.make_async_copy(dst_hbm.at[pl.ds(b, CHUNK)],
                                  dsti.at[slot], dis.at[slot]).wait()
            pltpu.async_copy(rows[bank], acc.at[dsti.at[slot]],
                             ssem.at[bank], add=True)
        return 0

    lax.fori_loop(0, CH_PER_W // 4, body, 0)
    pltpu.make_async_copy(rows[1], acc.at[dsti.at[3]], ssem.at[1]).wait()
    plsc.subcore_barrier()

    wd = []
    for t in range(ROWS_PER_TILE // CHUNK):
        k = t % NSETS
        r0 = s * ROWS_PER_TILE + t * CHUNK
        if t >= NSETS:
            wd[t - NSETS].wait()
        pltpu.sync_copy(acc.at[pl.ds(r0, CHUNK)], rows[k])
        wd.append(pltpu.async_copy(rows[k], out_hbm.at[c, pl.ds(r0, CHUNK)],
                                   ssem.at[k]))
    for t in range(max(0, ROWS_PER_TILE // CHUNK - NSETS),
                   ROWS_PER_TILE // CHUNK):
        wd[t].wait()


_seg_sum = pl.kernel(
    _seg_sum_body,
    out_type=jax.ShapeDtypeStruct((NC, N_PAD, D), jnp.float32),
    mesh=plsc.VectorSubcoreMesh(core_axis_name="c", subcore_axis_name="s"),
    scratch_types=[
        pltpu.VMEM((2 * NSETS, CHUNK), jnp.int32),
        pltpu.VMEM((2 * NSETS, CHUNK), jnp.int32),
        pltpu.VMEM((CHUNK, D), jnp.float32),
        pltpu.VMEM((CHUNK, D), jnp.float32),
        pltpu.VMEM_SHARED((N_PAD, D), jnp.float32),
        pltpu.SemaphoreType.DMA((2 * NSETS,)),
        pltpu.SemaphoreType.DMA((2 * NSETS,)),
        pltpu.SemaphoreType.DMA((NSETS,)),
        pltpu.SemaphoreType.DMA((NSETS,)),
        pltpu.SemaphoreType.DMA,
    ],
)


def _layer1_body(p_ref, W_ref, b_ref, o_ref):
    a = p_ref[0] + p_ref[1]
    o_ref[...] = jnp.maximum(
        jnp.dot(a, W_ref[...], preferred_element_type=jnp.float32)
        + b_ref[...], 0.0)


def _layer2_body(p_ref, W_ref, b_ref, hw_ref, hb_ref, o_ref):
    a = p_ref[0] + p_ref[1]
    h = jnp.maximum(
        jnp.dot(a, W_ref[...], preferred_element_type=jnp.float32)
        + b_ref[...], 0.0)
    o_ref[...] = jnp.dot(h, hw_ref[...],
                         preferred_element_type=jnp.float32) + hb_ref[...]


_BN = 1024


def _tc_layer1(p, W, b):
    return pl.pallas_call(
        _layer1_body,
        grid=(N_PAD // _BN,),
        in_specs=[
            pl.BlockSpec((NC, _BN, D), lambda i: (0, i, 0)),
            pl.BlockSpec((D, D), lambda i: (0, 0)),
            pl.BlockSpec((1, D), lambda i: (0, 0)),
        ],
        out_specs=pl.BlockSpec((_BN, D), lambda i: (i, 0)),
        out_shape=jax.ShapeDtypeStruct((N_PAD, D), jnp.float32),
    )(p, W, b)


def _tc_layer2(p, W, b, head_w, head_b):
    return pl.pallas_call(
        _layer2_body,
        grid=(N_PAD // _BN,),
        in_specs=[
            pl.BlockSpec((NC, _BN, D), lambda i: (0, i, 0)),
            pl.BlockSpec((D, D), lambda i: (0, 0)),
            pl.BlockSpec((1, D), lambda i: (0, 0)),
            pl.BlockSpec((D, 1), lambda i: (0, 0)),
            pl.BlockSpec((1, 1), lambda i: (0, 0)),
        ],
        out_specs=pl.BlockSpec((_BN, 1), lambda i: (i, 0)),
        out_shape=jax.ShapeDtypeStruct((N_PAD, 1), jnp.float32),
    )(p, W, b, head_w, head_b)


def kernel(x, edge_index, W1, b1, W2, b2, head_w, head_b):
    src = edge_index[0]
    dst = edge_index[1]
    pad = E_PAD - E
    it = lax.iota(jnp.int32, pad)
    src_p = jnp.concatenate([src, it % N_NODES])
    dst_p = jnp.concatenate([dst, N_NODES + it % (N_PAD - N_NODES)])

    agg1 = _seg_sum(x, src_p, dst_p)
    h1 = _tc_layer1(agg1, W1, b1.reshape(1, D))
    agg2 = _seg_sum(h1, src_p, dst_p)
    logits = _tc_layer2(agg2, W2, b2.reshape(1, D),
                        head_w, head_b.reshape(1, 1))
    return logits[:N_NODES, 0]


# speedup vs baseline: 12.4190x; 1.0603x over previous
import jax
import jax.numpy as jnp
from jax import lax
from jax.experimental import pallas as pl
from jax.experimental.pallas import tpu as pltpu
from jax.experimental.pallas import tpu_sc as plsc

N_NODES = 10000
N_PAD = 10240
D = 128
E = 320000
NC, NS = 2, 16
NW = NC * NS
CHUNK = 128
CH_TOT = E // CHUNK
CH_PER_W = 79
ROWS_PER_TILE = N_PAD // NS


def _seg_sum_body(x_hbm, ei_hbm, out_hbm,
                  srci, dsti, rows0, rows1, acc,
                  sis, dis, gsem, ssem, zsem):
    c = lax.axis_index("c")
    s = lax.axis_index("s")
    w = c * NS + s
    rows = [rows0, rows1]

    def chunk_base(j):
        cc = w + NW * j
        return pl.multiple_of(
            jnp.where(cc < CH_TOT, cc, w) * CHUNK, 8), cc

    zeros16 = jnp.zeros((16,), jnp.float32)

    def zero_rows(buf):
        def zbody(i, _):
            r = i // (D // 16)
            col = (i % (D // 16)) * 16
            buf[r, pl.ds(col, 16)] = zeros16
            return 0
        lax.fori_loop(0, CHUNK * (D // 16), zbody, 0)

    zero_rows(rows0)
    zd = [pltpu.async_copy(
        rows0, acc.at[pl.ds(s * ROWS_PER_TILE + t * CHUNK, CHUNK)], zsem)
        for t in range(ROWS_PER_TILE // CHUNK)]
    for d in zd:
        d.wait()
    plsc.subcore_barrier()

    def idx_issue(slot, j):
        b, _ = chunk_base(j)
        pltpu.async_copy(ei_hbm.at[0, pl.ds(b, CHUNK)], srci.at[slot],
                         sis.at[slot])
        pltpu.async_copy(ei_hbm.at[1, pl.ds(b, CHUNK)], dsti.at[slot],
                         dis.at[slot])

    def idx_wait_src(slot, j):
        b, _ = chunk_base(j)
        pltpu.make_async_copy(ei_hbm.at[0, pl.ds(b, CHUNK)], srci.at[slot],
                              sis.at[slot]).wait()

    def idx_wait_dst(slot, j):
        b, _ = chunk_base(j)
        pltpu.make_async_copy(ei_hbm.at[1, pl.ds(b, CHUNK)], dsti.at[slot],
                              dis.at[slot]).wait()

    def step(j, t, drain, prefetch, gather):
        bank = t % 2
        if drain is not False:
            def _drain():
                pltpu.make_async_copy(rows[1 - bank],
                                      acc.at[dsti.at[(t + 3) % 4]],
                                      ssem.at[1 - bank]).wait()
            if drain is True:
                _drain()
            else:
                pl.when(drain)(_drain)
        if prefetch:
            idx_issue((t + 2) % 4, j + 2)
        if gather:
            idx_wait_src((t + 1) % 4, j + 1)
            pltpu.async_copy(x_hbm.at[srci.at[(t + 1) % 4]],
                             rows[1 - bank], gsem.at[1 - bank])
        pltpu.make_async_copy(x_hbm.at[srci.at[t]], rows[bank],
                              gsem.at[bank]).wait()
        idx_wait_dst(t, j)
        _, cc = chunk_base(j)

        @pl.when(cc >= CH_TOT)
        def _():
            zero_rows(rows[bank])

        pltpu.async_copy(rows[bank], acc.at[dsti.at[t]],
                         ssem.at[bank], add=True)

    idx_issue(0, 0)
    idx_issue(1, 1)
    idx_wait_src(0, 0)
    pltpu.async_copy(x_hbm.at[srci.at[0]], rows[0], gsem.at[0])

    def body(i4, _):
        j0 = 4 * i4
        step(j0, 0, i4 > 0, True, True)
        step(j0 + 1, 1, True, True, True)
        step(j0 + 2, 2, True, True, True)
        step(j0 + 3, 3, True, True, True)
        return 0

    lax.fori_loop(0, CH_PER_W // 4, body, 0)
    step(76, 0, True, True, True)
    step(77, 1, True, False, True)
    step(78, 2, True, False, False)
    pltpu.make_async_copy(rows[0], acc.at[dsti.at[2]], ssem.at[0]).wait()
    plsc.subcore_barrier()

    wd = []
    for t in range(ROWS_PER_TILE // CHUNK):
        k = t % 2
        r0 = s * ROWS_PER_TILE + t * CHUNK
        if t >= 2:
            wd[t - 2].wait()
        pltpu.sync_copy(acc.at[pl.ds(r0, CHUNK)], rows[k])
        wd.append(pltpu.async_copy(rows[k], out_hbm.at[c, pl.ds(r0, CHUNK)],
                                   ssem.at[k]))
    for t in range(max(0, ROWS_PER_TILE // CHUNK - 2),
                   ROWS_PER_TILE // CHUNK):
        wd[t].wait()


_seg_sum = pl.kernel(
    _seg_sum_body,
    out_type=jax.ShapeDtypeStruct((NC, N_PAD, D), jnp.float32),
    mesh=plsc.VectorSubcoreMesh(core_axis_name="c", subcore_axis_name="s"),
    scratch_types=[
        pltpu.VMEM((4, CHUNK), jnp.int32),
        pltpu.VMEM((4, CHUNK), jnp.int32),
        pltpu.VMEM((CHUNK, D), jnp.float32),
        pltpu.VMEM((CHUNK, D), jnp.float32),
        pltpu.VMEM_SHARED((N_PAD, D), jnp.float32),
        pltpu.SemaphoreType.DMA((4,)),
        pltpu.SemaphoreType.DMA((4,)),
        pltpu.SemaphoreType.DMA((2,)),
        pltpu.SemaphoreType.DMA((2,)),
        pltpu.SemaphoreType.DMA,
    ],
)


def _layer1_body(p_ref, W_ref, b_ref, o_ref):
    a = p_ref[0] + p_ref[1]
    o_ref[...] = jnp.maximum(
        jnp.dot(a, W_ref[...], preferred_element_type=jnp.float32)
        + b_ref[...], 0.0)


def _layer2_body(p_ref, W_ref, b_ref, hw_ref, hb_ref, o_ref):
    a = p_ref[0] + p_ref[1]
    h = jnp.maximum(
        jnp.dot(a, W_ref[...], preferred_element_type=jnp.float32)
        + b_ref[...], 0.0)
    o_ref[...] = jnp.dot(h, hw_ref[...],
                         preferred_element_type=jnp.float32) + hb_ref[...]


def _tc_layer1(p, W, b):
    return pl.pallas_call(
        _layer1_body,
        out_shape=jax.ShapeDtypeStruct((N_PAD, D), jnp.float32),
    )(p, W, b)


def _tc_layer2(p, W, b, head_w, head_b):
    return pl.pallas_call(
        _layer2_body,
        out_shape=jax.ShapeDtypeStruct((N_PAD, 1), jnp.float32),
    )(p, W, b, head_w, head_b)


def kernel(x, edge_index, W1, b1, W2, b2, head_w, head_b):
    agg1 = _seg_sum(x, edge_index)
    h1 = _tc_layer1(agg1, W1, b1.reshape(1, D))
    agg2 = _seg_sum(h1, edge_index)
    logits = _tc_layer2(agg2, W2, b2.reshape(1, D),
                        head_w, head_b.reshape(1, 1))
    return logits[:N_NODES, 0]


# speedup vs baseline: 12.5560x; 1.0110x over previous
import jax
import jax.numpy as jnp
from jax import lax
from jax.experimental import pallas as pl
from jax.experimental.pallas import tpu as pltpu
from jax.experimental.pallas import tpu_sc as plsc

N_NODES = 10000
N_PAD = 10240
D = 128
E = 320000
NC, NS = 2, 16
NW = NC * NS
CHUNK = 128
CH_TOT = E // CHUNK
CH_PER_W = 79
ROWS_PER_TILE = N_PAD // NS


def _seg_sum_body(x_hbm, ei_hbm, out_hbm,
                  srci, dsti, rows0, rows1, acc,
                  sis, dis, gsem, ssem, zsem):
    c = lax.axis_index("c")
    s = lax.axis_index("s")
    w = c * NS + s
    rows = [rows0, rows1]

    def chunk_base(j):
        cc = w + NW * j
        return pl.multiple_of(
            jnp.where(cc < CH_TOT, cc, w) * CHUNK, 8), cc

    zeros16 = jnp.zeros((16,), jnp.float32)

    def zero_rows(buf):
        def zbody(i, _):
            r = i // (D // 16)
            col = (i % (D // 16)) * 16
            buf[r, pl.ds(col, 16)] = zeros16
            return 0
        lax.fori_loop(0, CHUNK * (D // 16), zbody, 0)

    zero_rows(rows0)
    zd = [pltpu.async_copy(
        rows0, acc.at[pl.ds(s * ROWS_PER_TILE + t * CHUNK, CHUNK)], zsem)
        for t in range(ROWS_PER_TILE // CHUNK)]
    for d in zd:
        d.wait()
    plsc.subcore_barrier()

    def idx_issue(slot, j):
        b, _ = chunk_base(j)
        pltpu.async_copy(ei_hbm.at[0, pl.ds(b, CHUNK)], srci.at[slot],
                         sis.at[slot])
        pltpu.async_copy(ei_hbm.at[1, pl.ds(b, CHUNK)], dsti.at[slot],
                         dis.at[slot])

    def idx_wait_src(slot, j):
        b, _ = chunk_base(j)
        pltpu.make_async_copy(ei_hbm.at[0, pl.ds(b, CHUNK)], srci.at[slot],
                              sis.at[slot]).wait()

    def idx_wait_dst(slot, j):
        b, _ = chunk_base(j)
        pltpu.make_async_copy(ei_hbm.at[1, pl.ds(b, CHUNK)], dsti.at[slot],
                              dis.at[slot]).wait()

    def step(j, t, drain, prefetch, gather):
        bank = t % 2
        if drain is not False:
            def _drain():
                pltpu.make_async_copy(rows[1 - bank],
                                      acc.at[dsti.at[(t + 3) % 4]],
                                      ssem.at[1 - bank]).wait()
            if drain is True:
                _drain()
            else:
                pl.when(drain)(_drain)
        if prefetch:
            idx_issue((t + 2) % 4, j + 2)
        if gather:
            idx_wait_src((t + 1) % 4, j + 1)
            pltpu.async_copy(x_hbm.at[srci.at[(t + 1) % 4]],
                             rows[1 - bank], gsem.at[1 - bank])
        pltpu.make_async_copy(x_hbm.at[srci.at[t]], rows[bank],
                              gsem.at[bank]).wait()
        idx_wait_dst(t, j)
        _, cc = chunk_base(j)

        @pl.when(cc >= CH_TOT)
        def _():
            zero_rows(rows[bank])

        pltpu.async_copy(rows[bank], acc.at[dsti.at[t]],
                         ssem.at[bank], add=True)

    idx_issue(0, 0)
    idx_issue(1, 1)
    idx_wait_src(0, 0)
    pltpu.async_copy(x_hbm.at[srci.at[0]], rows[0], gsem.at[0])

    def body(i4, _):
        j0 = 4 * i4
        step(j0, 0, i4 > 0, True, True)
        step(j0 + 1, 1, True, True, True)
        step(j0 + 2, 2, True, True, True)
        step(j0 + 3, 3, True, True, True)
        return 0

    lax.fori_loop(0, CH_PER_W // 4, body, 0)
    step(76, 0, True, True, True)
    step(77, 1, True, False, True)
    step(78, 2, True, False, False)
    pltpu.make_async_copy(rows[0], acc.at[dsti.at[2]], ssem.at[0]).wait()
    plsc.subcore_barrier()

    wd = []
    for t in range(ROWS_PER_TILE // CHUNK):
        k = t % 2
        r0 = s * ROWS_PER_TILE + t * CHUNK
        if t >= 2:
            wd[t - 2].wait()
        pltpu.sync_copy(acc.at[pl.ds(r0, CHUNK)], rows[k])
        wd.append(pltpu.async_copy(rows[k], out_hbm.at[c, pl.ds(r0, CHUNK)],
                                   ssem.at[k]))
    for t in range(max(0, ROWS_PER_TILE // CHUNK - 2),
                   ROWS_PER_TILE // CHUNK):
        wd[t].wait()


_seg_sum = pl.kernel(
    _seg_sum_body,
    out_type=jax.ShapeDtypeStruct((NC, N_PAD, D), jnp.float32),
    mesh=plsc.VectorSubcoreMesh(core_axis_name="c", subcore_axis_name="s"),
    scratch_types=[
        pltpu.VMEM((4, CHUNK), jnp.int32),
        pltpu.VMEM((4, CHUNK), jnp.int32),
        pltpu.VMEM((CHUNK, D), jnp.float32),
        pltpu.VMEM((CHUNK, D), jnp.float32),
        pltpu.VMEM_SHARED((N_PAD, D), jnp.float32),
        pltpu.SemaphoreType.DMA((4,)),
        pltpu.SemaphoreType.DMA((4,)),
        pltpu.SemaphoreType.DMA((2,)),
        pltpu.SemaphoreType.DMA((2,)),
        pltpu.SemaphoreType.DMA,
    ],
)


def _layer1_body(p_ref, W_ref, b_ref, o_ref):
    a = p_ref[0] + p_ref[1]
    o_ref[...] = jnp.maximum(
        jnp.dot(a, W_ref[...], preferred_element_type=jnp.float32)
        + b_ref[...], 0.0)


def _layer2_body(p_ref, W_ref, b_ref, hw_ref, hb_ref, o_ref):
    a = p_ref[0] + p_ref[1]
    h = jnp.maximum(
        jnp.dot(a, W_ref[...], preferred_element_type=jnp.float32)
        + b_ref[...], 0.0)
    lg = jnp.dot(h, hw_ref[...],
                 preferred_element_type=jnp.float32) + hb_ref[...]
    o_ref[...] = lg[:N_NODES, 0]


def _tc_layer1(p, W, b):
    return pl.pallas_call(
        _layer1_body,
        out_shape=jax.ShapeDtypeStruct((N_PAD, D), jnp.float32),
    )(p, W, b)


def _tc_layer2(p, W, b, head_w, head_b):
    return pl.pallas_call(
        _layer2_body,
        out_shape=jax.ShapeDtypeStruct((N_NODES,), jnp.float32),
    )(p, W, b, head_w, head_b)


def kernel(x, edge_index, W1, b1, W2, b2, head_w, head_b):
    agg1 = _seg_sum(x, edge_index)
    h1 = _tc_layer1(agg1, W1, b1.reshape(1, D))
    agg2 = _seg_sum(h1, edge_index)
    return _tc_layer2(agg2, W2, b2.reshape(1, D),
                      head_w, head_b.reshape(1, 1))
